# Initial kernel scaffold; baseline (speedup 1.0000x reference)
import jax
import jax.numpy as jnp
from jax.experimental import pallas as pl


def kernel(x_s, x_t, edge_index, edge_attr, x_u, W1, b1, W2, b2, W3, b3, W4, b4, g):
    raise NotImplementedError("write your pallas kernel here")


# speedup vs baseline: 2.8953x; 2.8953x over previous
import functools

import jax
import jax.numpy as jnp
from jax import lax
from jax.experimental import pallas as pl
from jax.experimental.pallas import tpu as pltpu
from jax.experimental.pallas import tpu_sc as plsc

N = 10000
E = 320000
DS = 128
DT = 128
DE = 16
M = 144
MP = 160
GW = 128
U = DS + 4 * M + GW
UP = DS + 4 * MP + GW

NC = 2
NS = 16
NW = NC * NS
LANES = 16

LEAKY = 0.2
F32EPS = 1.1920929e-07


def _leaky(x):
    return jnp.where(x >= 0, x, LEAKY * x)


_RB = 512


def _edge_mlp_body(g_ref, ea_ref, w1a_ref, w1b_ref, b1_ref, w2_ref, b2_ref,
                   out_ref):
    z = (
        jnp.dot(g_ref[...], w1a_ref[...], preferred_element_type=jnp.float32)
        + jnp.dot(ea_ref[...], w1b_ref[...], preferred_element_type=jnp.float32)
        + b1_ref[...]
    )
    a = _leaky(z)
    out_ref[...] = (
        jnp.dot(a, w2_ref[...], preferred_element_type=jnp.float32)
        + b2_ref[...]
    )


def _edge_mlp(g, edge_attr, W1a, W1b, b1, W2p, b2p):
    grid = (E // _RB,)
    return pl.pallas_call(
        _edge_mlp_body,
        grid=grid,
        in_specs=[
            pl.BlockSpec((_RB, DT), lambda i: (i, 0)),
            pl.BlockSpec((_RB, DE), lambda i: (i, 0)),
            pl.BlockSpec((DT, M), lambda i: (0, 0)),
            pl.BlockSpec((DE, M), lambda i: (0, 0)),
            pl.BlockSpec((1, M), lambda i: (0, 0)),
            pl.BlockSpec((M, MP), lambda i: (0, 0)),
            pl.BlockSpec((1, MP), lambda i: (0, 0)),
        ],
        out_specs=pl.BlockSpec((_RB, MP), lambda i: (i, 0)),
        out_shape=jax.ShapeDtypeStruct((E, MP), jnp.float32),
    )(g, edge_attr, W1a, W1b, b1, W2p, b2p)


_NB = 1000


def _node_mlp_body(xs_ref, s1_ref, s2_ref, s3_ref, s4_ref, xu_ref,
                   w3_ref, b3_ref, w4_ref, b4_ref, g_ref, out_ref):
    s1 = s1_ref[0] + s1_ref[1]
    cmask = lax.broadcasted_iota(jnp.int32, (_NB, MP), 1) == M
    c = jnp.maximum(
        jnp.sum(jnp.where(cmask, s1, 0.0), axis=-1, keepdims=True), 1.0)
    s2 = s2_ref[0] + s2_ref[1]
    s3 = s3_ref[0] + s3_ref[1]
    s4 = s4_ref[0] + s4_ref[1]
    mean = s1 / c
    m2 = s2 / c
    m3 = s3 / c
    m4 = s4 / c
    mm = mean * mean
    var = _leaky(m2 - mm)
    s2v = var + 1e-6
    std = jnp.sqrt(s2v)
    m3c = m3 - 3.0 * mean * m2 + 2.0 * mean * mm
    m4c = m4 - 4.0 * mean * m3 + 6.0 * mm * m2 - 3.0 * mm * mm
    skew = m3c / (s2v * std)
    kurt = m4c / (s2v * s2v)

    t = (
        jnp.dot(xs_ref[...], w3_ref[0:DS, :], preferred_element_type=jnp.float32)
        + jnp.dot(mean, w3_ref[DS:DS + MP, :], preferred_element_type=jnp.float32)
        + jnp.dot(std, w3_ref[DS + MP:DS + 2 * MP, :], preferred_element_type=jnp.float32)
        + jnp.dot(skew, w3_ref[DS + 2 * MP:DS + 3 * MP, :], preferred_element_type=jnp.float32)
        + jnp.dot(kurt, w3_ref[DS + 3 * MP:DS + 4 * MP, :], preferred_element_type=jnp.float32)
        + jnp.dot(xu_ref[...], w3_ref[DS + 4 * MP:UP, :], preferred_element_type=jnp.float32)
        + b3_ref[...]
    )
    a = _leaky(t)
    out = jnp.dot(a, w4_ref[...], preferred_element_type=jnp.float32) + b4_ref[...]
    r = out * lax.rsqrt(jnp.mean(out * out, axis=-1, keepdims=True) + F32EPS)
    out_ref[...] = r * g_ref[...]


def _node_mlp(x_s, S1, S2, S3, S4, x_u, W3p, b3, W4, b4, g):
    grid = (N // _NB,)
    sspec = pl.BlockSpec((NC, _NB, MP), lambda i: (0, i, 0))
    return pl.pallas_call(
        _node_mlp_body,
        grid=grid,
        in_specs=[
            pl.BlockSpec((_NB, DS), lambda i: (i, 0)),
            sspec, sspec, sspec, sspec,
            pl.BlockSpec((1, GW), lambda i: (0, 0)),
            pl.BlockSpec((UP, U), lambda i: (0, 0)),
            pl.BlockSpec((1, U), lambda i: (0, 0)),
            pl.BlockSpec((U, DS), lambda i: (0, 0)),
            pl.BlockSpec((1, DS), lambda i: (0, 0)),
            pl.BlockSpec((1, DS), lambda i: (0, 0)),
        ],
        out_specs=pl.BlockSpec((_NB, DS), lambda i: (i, 0)),
        out_shape=jax.ShapeDtypeStruct((N, DS), jnp.float32),
    )(x_s, S1, S2, S3, S4, x_u, W3p, b3, W4, b4, g)


_CK = 80
_EPW = E // NW
_NCH = _EPW // _CK

_mesh = plsc.VectorSubcoreMesh(
    core_axis_name="c", subcore_axis_name="s", num_cores=NC, num_subcores=NS)


@functools.partial(
    pl.kernel,
    out_type=jax.ShapeDtypeStruct((E, DT), jnp.float32),
    mesh=_mesh,
    scratch_types=[
        pltpu.VMEM((_NCH, _CK), jnp.int32),
        pltpu.VMEM((_CK, DT), jnp.float32),
        pltpu.VMEM((_CK, DT), jnp.float32),
        pltpu.SemaphoreType.DMA,
        pltpu.SemaphoreType.DMA,
    ],
)
def _sc_gather(yt_hbm, tgt_hbm, out_hbm, idx_v, rows_a, rows_b, sem_a, sem_b):
    wid = lax.axis_index("c") * NS + lax.axis_index("s")
    base = wid * _EPW
    pltpu.sync_copy(tgt_hbm.at[wid], idx_v)

    bufs = (rows_a, rows_b)
    sems = (sem_a, sem_b)

    def start(i, b):
        pltpu.async_copy(yt_hbm.at[idx_v.at[i]], bufs[b], sems[b])

    def wait(i, b):
        pltpu.make_async_copy(yt_hbm.at[idx_v.at[i]], bufs[b], sems[b]).wait()

    def store(i, b):
        pltpu.sync_copy(bufs[b], out_hbm.at[pl.ds(base + i * _CK, _CK)])

    start(0, 0)

    def body(j, _):
        i0 = 2 * j
        wait(i0, 0)
        start(i0 + 1, 1)
        store(i0, 0)
        i1 = i0 + 1
        wait(i1, 1)

        @pl.when(i1 + 1 < _NCH)
        def _():
            start(i1 + 1, 0)

        store(i1, 1)
        return 0

    lax.fori_loop(0, (_NCH - 1) // 2, body, 0)
    wait(_NCH - 1, 0)
    store(_NCH - 1, 0)


_PC = 80
_NPIECE = N // _PC
_TP = (_NPIECE + NS - 1) // NS


@functools.partial(
    pl.kernel,
    out_type=(
        jax.ShapeDtypeStruct((NC, N, MP), jnp.float32),
        jax.ShapeDtypeStruct((NC, N, MP), jnp.float32),
        jax.ShapeDtypeStruct((NC, N, MP), jnp.float32),
        jax.ShapeDtypeStruct((NC, N, MP), jnp.float32),
    ),
    mesh=_mesh,
    scratch_types=[
        pltpu.VMEM((_NCH, _CK), jnp.int32),
        pltpu.VMEM((_CK, MP), jnp.float32),
        pltpu.VMEM_SHARED((N, MP), jnp.float32),
    ],
    compiler_params=pltpu.CompilerParams(use_tc_tiling_on_sc=False),
)
def _sc_moments(msg_hbm, src_hbm, zero_hbm, s1_hbm, s2_hbm, s3_hbm, s4_hbm,
                idx_v, rows_v, acc):
    cid = lax.axis_index("c")
    sid = lax.axis_index("s")
    wid = cid * NS + sid
    ebase = wid * _EPW
    souts = (s1_hbm, s2_hbm, s3_hbm, s4_hbm)

    pltpu.sync_copy(src_hbm.at[wid], idx_v)

    for p in range(4):
        for t in range(_TP):
            q = sid + NS * t

            @pl.when(q < _NPIECE)
            def _(q=q):
                pltpu.sync_copy(zero_hbm, acc.at[pl.ds(q * _PC, _PC)])

        plsc.subcore_barrier()

        def chunk(i, _, pass_id=p):
            pltpu.sync_copy(msg_hbm.at[pl.ds(ebase + i * _CK, _CK)], rows_v)

            if pass_id > 0:
                def powers(r, __):
                    for gix in range(MP // LANES):
                        sl = pl.ds(gix * LANES, LANES)
                        v = rows_v[r, sl]
                        v2 = v * v
                        if pass_id == 1:
                            rows_v[r, sl] = v2
                        elif pass_id == 2:
                            rows_v[r, sl] = v2 * v
                        else:
                            rows_v[r, sl] = v2 * v2
                    return 0

                lax.fori_loop(0, _CK, powers, 0)

            pltpu.sync_copy(rows_v, acc.at[idx_v.at[i]], add=True)
            return 0

        lax.fori_loop(0, _NCH, chunk, 0)
        plsc.subcore_barrier()

        for t in range(_TP):
            q = sid + NS * t

            @pl.when(q < _NPIECE)
            def _(q=q, p=p):
                rs = pl.ds(q * _PC, _PC)
                pltpu.sync_copy(acc.at[rs], souts[p].at[cid, rs])

        plsc.subcore_barrier()


def kernel(x_s, x_t, edge_index, edge_attr, x_u, W1, b1, W2, b2, W3, b3, W4,
           b4, g):
    src = edge_index[0].astype(jnp.int32).reshape(NW, _NCH, _CK)
    tgt = edge_index[1].astype(jnp.int32).reshape(NW, _NCH, _CK)

    W1a = W1[:DT]
    W1b = W1[DT:]
    W2p = jnp.pad(W2, ((0, 0), (0, MP - M)))
    b2p = jnp.concatenate(
        [b2, jnp.ones((1,), jnp.float32),
         jnp.zeros((MP - M - 1,), jnp.float32)]).reshape(1, MP)
    z16 = jnp.zeros((MP - M, U), jnp.float32)
    W3p = jnp.concatenate([
        W3[0:DS],
        W3[DS:DS + M], z16,
        W3[DS + M:DS + 2 * M], z16,
        W3[DS + 2 * M:DS + 3 * M], z16,
        W3[DS + 3 * M:DS + 4 * M], z16,
        W3[DS + 4 * M:U],
    ], axis=0)

    gathered = _sc_gather(x_t, tgt)
    msg = _edge_mlp(gathered, edge_attr, W1a, W1b, b1.reshape(1, M),
                    W2p, b2p)
    S1, S2, S3, S4 = _sc_moments(msg, src, jnp.zeros((_PC, MP), jnp.float32))
    return _node_mlp(x_s, S1, S2, S3, S4, x_u.reshape(1, GW),
                     W3p, b3.reshape(1, U), W4, b4.reshape(1, DS),
                     g.reshape(1, DS))


# speedup vs baseline: 3.4906x; 1.2056x over previous
import functools

import jax
import jax.numpy as jnp
from jax import lax
from jax.experimental import pallas as pl
from jax.experimental.pallas import tpu as pltpu
from jax.experimental.pallas import tpu_sc as plsc

N = 10000
E = 320000
DS = 128
DT = 128
DE = 16
M = 144
MP = 160
GW = 128
U = DS + 4 * M + GW
UP = DS + 4 * MP + GW

NC = 2
NS = 16
NW = NC * NS
LANES = 16

LEAKY = 0.2
F32EPS = 1.1920929e-07


def _leaky(x):
    return jnp.where(x >= 0, x, LEAKY * x)


_RB = 512


def _edge_mlp_body(g_ref, ea_ref, w1a_ref, w1b_ref, b1_ref, w2_ref, b2_ref,
                   out_ref):
    z = (
        jnp.dot(g_ref[...], w1a_ref[...], preferred_element_type=jnp.float32)
        + jnp.dot(ea_ref[...], w1b_ref[...], preferred_element_type=jnp.float32)
        + b1_ref[...]
    )
    a = _leaky(z)
    out_ref[...] = (
        jnp.dot(a, w2_ref[...], preferred_element_type=jnp.float32)
        + b2_ref[...]
    )


def _edge_mlp(g, edge_attr, W1a, W1b, b1, W2p, b2p):
    grid = (E // _RB,)
    return pl.pallas_call(
        _edge_mlp_body,
        grid=grid,
        in_specs=[
            pl.BlockSpec((_RB, DT), lambda i: (i, 0)),
            pl.BlockSpec((_RB, DE), lambda i: (i, 0)),
            pl.BlockSpec((DT, M), lambda i: (0, 0)),
            pl.BlockSpec((DE, M), lambda i: (0, 0)),
            pl.BlockSpec((1, M), lambda i: (0, 0)),
            pl.BlockSpec((M, MP), lambda i: (0, 0)),
            pl.BlockSpec((1, MP), lambda i: (0, 0)),
        ],
        out_specs=pl.BlockSpec((_RB, MP), lambda i: (i, 0)),
        out_shape=jax.ShapeDtypeStruct((E, MP), jnp.float32),
    )(g, edge_attr, W1a, W1b, b1, W2p, b2p)


_NB = 1000


def _node_mlp_body(xs_ref, s1_ref, s2_ref, s3_ref, s4_ref, xu_ref,
                   w3_ref, b3_ref, w4_ref, b4_ref, g_ref, out_ref):
    s1 = s1_ref[0] + s1_ref[1]
    cmask = lax.broadcasted_iota(jnp.int32, (_NB, MP), 1) == M
    c = jnp.maximum(
        jnp.sum(jnp.where(cmask, s1, 0.0), axis=-1, keepdims=True), 1.0)
    s2 = s2_ref[0] + s2_ref[1]
    s3 = s3_ref[0] + s3_ref[1]
    s4 = s4_ref[0] + s4_ref[1]
    mean = s1 / c
    m2 = s2 / c
    m3 = s3 / c
    m4 = s4 / c
    mm = mean * mean
    var = _leaky(m2 - mm)
    s2v = var + 1e-6
    std = jnp.sqrt(s2v)
    m3c = m3 - 3.0 * mean * m2 + 2.0 * mean * mm
    m4c = m4 - 4.0 * mean * m3 + 6.0 * mm * m2 - 3.0 * mm * mm
    skew = m3c / (s2v * std)
    kurt = m4c / (s2v * s2v)

    t = (
        jnp.dot(xs_ref[...], w3_ref[0:DS, :], preferred_element_type=jnp.float32)
        + jnp.dot(mean, w3_ref[DS:DS + MP, :], preferred_element_type=jnp.float32)
        + jnp.dot(std, w3_ref[DS + MP:DS + 2 * MP, :], preferred_element_type=jnp.float32)
        + jnp.dot(skew, w3_ref[DS + 2 * MP:DS + 3 * MP, :], preferred_element_type=jnp.float32)
        + jnp.dot(kurt, w3_ref[DS + 3 * MP:DS + 4 * MP, :], preferred_element_type=jnp.float32)
        + jnp.dot(xu_ref[...], w3_ref[DS + 4 * MP:UP, :], preferred_element_type=jnp.float32)
        + b3_ref[...]
    )
    a = _leaky(t)
    out = jnp.dot(a, w4_ref[...], preferred_element_type=jnp.float32) + b4_ref[...]
    r = out * lax.rsqrt(jnp.mean(out * out, axis=-1, keepdims=True) + F32EPS)
    out_ref[...] = r * g_ref[...]


def _node_mlp(x_s, S1, S2, S3, S4, x_u, W3p, b3, W4, b4, g):
    grid = (N // _NB,)
    sspec = pl.BlockSpec((NC, _NB, MP), lambda i: (0, i, 0))
    return pl.pallas_call(
        _node_mlp_body,
        grid=grid,
        in_specs=[
            pl.BlockSpec((_NB, DS), lambda i: (i, 0)),
            sspec, sspec, sspec, sspec,
            pl.BlockSpec((1, GW), lambda i: (0, 0)),
            pl.BlockSpec((UP, U), lambda i: (0, 0)),
            pl.BlockSpec((1, U), lambda i: (0, 0)),
            pl.BlockSpec((U, DS), lambda i: (0, 0)),
            pl.BlockSpec((1, DS), lambda i: (0, 0)),
            pl.BlockSpec((1, DS), lambda i: (0, 0)),
        ],
        out_specs=pl.BlockSpec((_NB, DS), lambda i: (i, 0)),
        out_shape=jax.ShapeDtypeStruct((N, DS), jnp.float32),
    )(x_s, S1, S2, S3, S4, x_u, W3p, b3, W4, b4, g)


_CK = 80
_EPW = E // NW
_NCH = _EPW // _CK

_mesh = plsc.VectorSubcoreMesh(
    core_axis_name="c", subcore_axis_name="s", num_cores=NC, num_subcores=NS)


@functools.partial(
    pl.kernel,
    out_type=jax.ShapeDtypeStruct((E, DT), jnp.float32),
    mesh=_mesh,
    scratch_types=[
        pltpu.VMEM((_NCH, _CK), jnp.int32),
        pltpu.VMEM((_CK, DT), jnp.float32),
        pltpu.VMEM((_CK, DT), jnp.float32),
        pltpu.SemaphoreType.DMA,
        pltpu.SemaphoreType.DMA,
    ],
)
def _sc_gather(yt_hbm, tgt_hbm, out_hbm, idx_v, rows_a, rows_b, sem_a, sem_b):
    wid = lax.axis_index("c") * NS + lax.axis_index("s")
    base = wid * _EPW
    pltpu.sync_copy(tgt_hbm.at[wid], idx_v)

    bufs = (rows_a, rows_b)
    sems = (sem_a, sem_b)

    def start(i, b):
        pltpu.async_copy(yt_hbm.at[idx_v.at[i]], bufs[b], sems[b])

    def wait(i, b):
        pltpu.make_async_copy(yt_hbm.at[idx_v.at[i]], bufs[b], sems[b]).wait()

    def store(i, b):
        pltpu.sync_copy(bufs[b], out_hbm.at[pl.ds(base + i * _CK, _CK)])

    start(0, 0)

    def body(j, _):
        i0 = 2 * j
        wait(i0, 0)
        start(i0 + 1, 1)
        store(i0, 0)
        i1 = i0 + 1
        wait(i1, 1)

        @pl.when(i1 + 1 < _NCH)
        def _():
            start(i1 + 1, 0)

        store(i1, 1)
        return 0

    lax.fori_loop(0, (_NCH - 1) // 2, body, 0)
    wait(_NCH - 1, 0)
    store(_NCH - 1, 0)


_PC = 80
_NPIECE = N // _PC
_TP = (_NPIECE + NS - 1) // NS


@functools.partial(
    pl.kernel,
    out_type=(
        jax.ShapeDtypeStruct((NC, N, MP), jnp.float32),
        jax.ShapeDtypeStruct((NC, N, MP), jnp.float32),
        jax.ShapeDtypeStruct((NC, N, MP), jnp.float32),
        jax.ShapeDtypeStruct((NC, N, MP), jnp.float32),
    ),
    mesh=_mesh,
    scratch_types=[
        pltpu.VMEM((_CK,), jnp.int32),
        pltpu.VMEM((_CK,), jnp.int32),
        pltpu.VMEM((_CK, MP), jnp.float32),
        pltpu.VMEM((_CK, MP), jnp.float32),
        pltpu.VMEM_SHARED((N, MP), jnp.float32),
        pltpu.SemaphoreType.DMA,
        pltpu.SemaphoreType.DMA,
        pltpu.SemaphoreType.DMA,
        pltpu.SemaphoreType.DMA,
        pltpu.SemaphoreType.DMA,
        pltpu.SemaphoreType.DMA,
    ],
    compiler_params=pltpu.CompilerParams(use_tc_tiling_on_sc=False),
)
def _sc_moments(msg_hbm, src_hbm, zero_hbm, s1_hbm, s2_hbm, s3_hbm, s4_hbm,
                idx_a, idx_b, rows_a, rows_b, acc,
                smi_a, smi_b, smr_a, smr_b, sms_a, sms_b):
    cid = lax.axis_index("c")
    sid = lax.axis_index("s")
    wid = cid * NS + sid
    ebase = wid * _EPW
    souts = (s1_hbm, s2_hbm, s3_hbm, s4_hbm)

    idxs = (idx_a, idx_b)
    rows = (rows_a, rows_b)
    smi = (smi_a, smi_b)
    smr = (smr_a, smr_b)
    sms = (sms_a, sms_b)

    def load_start(i, b):
        pltpu.async_copy(src_hbm.at[wid, i], idxs[b], smi[b])
        pltpu.async_copy(msg_hbm.at[pl.ds(ebase + i * _CK, _CK)], rows[b],
                         smr[b])

    def load_wait(i, b):
        pltpu.make_async_copy(src_hbm.at[wid, i], idxs[b], smi[b]).wait()
        pltpu.make_async_copy(msg_hbm.at[pl.ds(ebase + i * _CK, _CK)],
                              rows[b], smr[b]).wait()

    def scat_start(b):
        pltpu.async_copy(rows[b], acc.at[idxs[b]], sms[b], add=True)

    def scat_wait(b):
        pltpu.make_async_copy(rows[b], acc.at[idxs[b]], sms[b]).wait()

    for p in range(4):
        for t in range(_TP):
            q = sid + NS * t

            @pl.when(q < _NPIECE)
            def _(q=q):
                pltpu.sync_copy(zero_hbm, acc.at[pl.ds(q * _PC, _PC)])

        plsc.subcore_barrier()

        def powers(buf, pass_id):
            if pass_id == 0:
                return

            def body(r, __):
                for gix in range(MP // LANES):
                    sl = pl.ds(gix * LANES, LANES)
                    v = buf[r, sl]
                    v2 = v * v
                    if pass_id == 1:
                        buf[r, sl] = v2
                    elif pass_id == 2:
                        buf[r, sl] = v2 * v
                    else:
                        buf[r, sl] = v2 * v2
                return 0

            lax.fori_loop(0, _CK, body, 0)

        load_start(0, 0)

        def pair(j, _, pass_id=p):
            i0 = 2 * j
            load_wait(i0, 0)

            @pl.when(j > 0)
            def _():
                scat_wait(1)

            load_start(i0 + 1, 1)
            powers(rows[0], pass_id)
            scat_start(0)

            i1 = i0 + 1
            load_wait(i1, 1)
            scat_wait(0)

            @pl.when(i1 + 1 < _NCH)
            def _():
                load_start(i1 + 1, 0)

            powers(rows[1], pass_id)
            scat_start(1)
            return 0

        lax.fori_loop(0, (_NCH - 1) // 2, pair, 0)
        i_last = _NCH - 1
        load_wait(i_last, 0)
        scat_wait(1)
        powers(rows[0], p)
        scat_start(0)
        scat_wait(0)
        plsc.subcore_barrier()

        for t in range(_TP):
            q = sid + NS * t

            @pl.when(q < _NPIECE)
            def _(q=q, p=p):
                rs = pl.ds(q * _PC, _PC)
                pltpu.sync_copy(acc.at[rs], souts[p].at[cid, rs])

        plsc.subcore_barrier()


def kernel(x_s, x_t, edge_index, edge_attr, x_u, W1, b1, W2, b2, W3, b3, W4,
           b4, g):
    src = edge_index[0].astype(jnp.int32).reshape(NW, _NCH, _CK)
    tgt = edge_index[1].astype(jnp.int32).reshape(NW, _NCH, _CK)

    W1a = W1[:DT]
    W1b = W1[DT:]
    W2p = jnp.pad(W2, ((0, 0), (0, MP - M)))
    b2p = jnp.concatenate(
        [b2, jnp.ones((1,), jnp.float32),
         jnp.zeros((MP - M - 1,), jnp.float32)]).reshape(1, MP)
    z16 = jnp.zeros((MP - M, U), jnp.float32)
    W3p = jnp.concatenate([
        W3[0:DS],
        W3[DS:DS + M], z16,
        W3[DS + M:DS + 2 * M], z16,
        W3[DS + 2 * M:DS + 3 * M], z16,
        W3[DS + 3 * M:DS + 4 * M], z16,
        W3[DS + 4 * M:U],
    ], axis=0)

    gathered = _sc_gather(x_t, tgt)
    msg = _edge_mlp(gathered, edge_attr, W1a, W1b, b1.reshape(1, M),
                    W2p, b2p)
    S1, S2, S3, S4 = _sc_moments(msg, src, jnp.zeros((_PC, MP), jnp.float32))
    return _node_mlp(x_s, S1, S2, S3, S4, x_u.reshape(1, GW),
                     W3p, b3.reshape(1, U), W4, b4.reshape(1, DS),
                     g.reshape(1, DS))


# speedup vs baseline: 4.0764x; 1.1678x over previous
import functools

import jax
import jax.numpy as jnp
from jax import lax
from jax.experimental import pallas as pl
from jax.experimental.pallas import tpu as pltpu
from jax.experimental.pallas import tpu_sc as plsc

N = 10000
E = 320000
DS = 128
DT = 128
DE = 16
M = 144
MP = 160
GW = 128
U = DS + 4 * M + GW
UP = DS + 4 * MP + GW

NC = 2
NS = 16
NW = NC * NS
LANES = 16

LEAKY = 0.2
F32EPS = 1.1920929e-07


def _leaky(x):
    return jnp.where(x >= 0, x, LEAKY * x)


_RB = 2048


def _edge_mlp_body(g_ref, ea_ref, w1a_ref, w1b_ref, b1_ref, w2_ref, b2_ref,
                   out_ref):
    z = (
        jnp.dot(g_ref[...], w1a_ref[...], preferred_element_type=jnp.float32)
        + jnp.dot(ea_ref[...], w1b_ref[...], preferred_element_type=jnp.float32)
        + b1_ref[...]
    )
    a = _leaky(z)
    out_ref[...] = (
        jnp.dot(a, w2_ref[...], preferred_element_type=jnp.float32)
        + b2_ref[...]
    )


def _edge_mlp(g, edge_attr, W1a, W1b, b1, W2p, b2p):
    grid = (E // _RB,)
    return pl.pallas_call(
        _edge_mlp_body,
        grid=grid,
        in_specs=[
            pl.BlockSpec((_RB, DT), lambda i: (i, 0)),
            pl.BlockSpec((_RB, DE), lambda i: (i, 0)),
            pl.BlockSpec((DT, M), lambda i: (0, 0)),
            pl.BlockSpec((DE, M), lambda i: (0, 0)),
            pl.BlockSpec((1, M), lambda i: (0, 0)),
            pl.BlockSpec((M, MP), lambda i: (0, 0)),
            pl.BlockSpec((1, MP), lambda i: (0, 0)),
        ],
        out_specs=pl.BlockSpec((_RB, MP), lambda i: (i, 0)),
        out_shape=jax.ShapeDtypeStruct((E, MP), jnp.float32),
    )(g, edge_attr, W1a, W1b, b1, W2p, b2p)


_NB = 1000


def _node_mlp_body(xs_ref, s1_ref, s2_ref, s3_ref, s4_ref, xu_ref,
                   w3_ref, b3_ref, w4_ref, b4_ref, g_ref, out_ref):
    s1 = s1_ref[0] + s1_ref[1]
    cmask = lax.broadcasted_iota(jnp.int32, (_NB, MP), 1) == M
    c = jnp.maximum(
        jnp.sum(jnp.where(cmask, s1, 0.0), axis=-1, keepdims=True), 1.0)
    s2 = s2_ref[0] + s2_ref[1]
    s3 = s3_ref[0] + s3_ref[1]
    s4 = s4_ref[0] + s4_ref[1]
    mean = s1 / c
    m2 = s2 / c
    m3 = s3 / c
    m4 = s4 / c
    mm = mean * mean
    var = _leaky(m2 - mm)
    s2v = var + 1e-6
    std = jnp.sqrt(s2v)
    m3c = m3 - 3.0 * mean * m2 + 2.0 * mean * mm
    m4c = m4 - 4.0 * mean * m3 + 6.0 * mm * m2 - 3.0 * mm * mm
    skew = m3c / (s2v * std)
    kurt = m4c / (s2v * s2v)

    t = (
        jnp.dot(xs_ref[...], w3_ref[0:DS, :], preferred_element_type=jnp.float32)
        + jnp.dot(mean, w3_ref[DS:DS + MP, :], preferred_element_type=jnp.float32)
        + jnp.dot(std, w3_ref[DS + MP:DS + 2 * MP, :], preferred_element_type=jnp.float32)
        + jnp.dot(skew, w3_ref[DS + 2 * MP:DS + 3 * MP, :], preferred_element_type=jnp.float32)
        + jnp.dot(kurt, w3_ref[DS + 3 * MP:DS + 4 * MP, :], preferred_element_type=jnp.float32)
        + jnp.dot(xu_ref[...], w3_ref[DS + 4 * MP:UP, :], preferred_element_type=jnp.float32)
        + b3_ref[...]
    )
    a = _leaky(t)
    out = jnp.dot(a, w4_ref[...], preferred_element_type=jnp.float32) + b4_ref[...]
    r = out * lax.rsqrt(jnp.mean(out * out, axis=-1, keepdims=True) + F32EPS)
    out_ref[...] = r * g_ref[...]


def _node_mlp(x_s, S1, S2, S3, S4, x_u, W3p, b3, W4, b4, g):
    grid = (N // _NB,)
    sspec = pl.BlockSpec((NC, _NB, MP), lambda i: (0, i, 0))
    return pl.pallas_call(
        _node_mlp_body,
        grid=grid,
        in_specs=[
            pl.BlockSpec((_NB, DS), lambda i: (i, 0)),
            sspec, sspec, sspec, sspec,
            pl.BlockSpec((1, GW), lambda i: (0, 0)),
            pl.BlockSpec((UP, U), lambda i: (0, 0)),
            pl.BlockSpec((1, U), lambda i: (0, 0)),
            pl.BlockSpec((U, DS), lambda i: (0, 0)),
            pl.BlockSpec((1, DS), lambda i: (0, 0)),
            pl.BlockSpec((1, DS), lambda i: (0, 0)),
        ],
        out_specs=pl.BlockSpec((_NB, DS), lambda i: (i, 0)),
        out_shape=jax.ShapeDtypeStruct((N, DS), jnp.float32),
    )(x_s, S1, S2, S3, S4, x_u, W3p, b3, W4, b4, g)


_CK = 80
_EPW = E // NW
_NCH = _EPW // _CK

_mesh = plsc.VectorSubcoreMesh(
    core_axis_name="c", subcore_axis_name="s", num_cores=NC, num_subcores=NS)


@functools.partial(
    pl.kernel,
    out_type=jax.ShapeDtypeStruct((E, DT), jnp.float32),
    mesh=_mesh,
    scratch_types=[
        pltpu.VMEM((_NCH, _CK), jnp.int32),
        pltpu.VMEM((_CK, DT), jnp.float32),
        pltpu.VMEM((_CK, DT), jnp.float32),
        pltpu.SemaphoreType.DMA,
        pltpu.SemaphoreType.DMA,
    ],
)
def _sc_gather(yt_hbm, tgt_hbm, out_hbm, idx_v, rows_a, rows_b, sem_a, sem_b):
    wid = lax.axis_index("c") * NS + lax.axis_index("s")
    base = wid * _EPW
    pltpu.sync_copy(tgt_hbm.at[wid], idx_v)

    bufs = (rows_a, rows_b)
    sems = (sem_a, sem_b)

    def start(i, b):
        pltpu.async_copy(yt_hbm.at[idx_v.at[i]], bufs[b], sems[b])

    def wait(i, b):
        pltpu.make_async_copy(yt_hbm.at[idx_v.at[i]], bufs[b], sems[b]).wait()

    def store(i, b):
        pltpu.sync_copy(bufs[b], out_hbm.at[pl.ds(base + i * _CK, _CK)])

    start(0, 0)

    def body(j, _):
        i0 = 2 * j
        wait(i0, 0)
        start(i0 + 1, 1)
        store(i0, 0)
        i1 = i0 + 1
        wait(i1, 1)

        @pl.when(i1 + 1 < _NCH)
        def _():
            start(i1 + 1, 0)

        store(i1, 1)
        return 0

    lax.fori_loop(0, (_NCH - 1) // 2, body, 0)
    wait(_NCH - 1, 0)
    store(_NCH - 1, 0)


_PC = 80
_NPIECE = N // _PC
_TP = (_NPIECE + NS - 1) // NS


@functools.partial(
    pl.kernel,
    out_type=(
        jax.ShapeDtypeStruct((NC, N, MP), jnp.float32),
        jax.ShapeDtypeStruct((NC, N, MP), jnp.float32),
        jax.ShapeDtypeStruct((NC, N, MP), jnp.float32),
        jax.ShapeDtypeStruct((NC, N, MP), jnp.float32),
    ),
    mesh=_mesh,
    scratch_types=[
        pltpu.VMEM((_CK,), jnp.int32),
        pltpu.VMEM((_CK,), jnp.int32),
        pltpu.VMEM((_CK, MP), jnp.float32),
        pltpu.VMEM((_CK, MP), jnp.float32),
        pltpu.VMEM_SHARED((N, MP), jnp.float32),
        pltpu.SemaphoreType.DMA,
        pltpu.SemaphoreType.DMA,
        pltpu.SemaphoreType.DMA,
        pltpu.SemaphoreType.DMA,
        pltpu.SemaphoreType.DMA,
        pltpu.SemaphoreType.DMA,
    ],
    compiler_params=pltpu.CompilerParams(use_tc_tiling_on_sc=False),
)
def _sc_moments(msg_hbm, src_hbm, zero_hbm, s1_hbm, s2_hbm, s3_hbm, s4_hbm,
                idx_a, idx_b, rows_a, rows_b, acc,
                smi_a, smi_b, smr_a, smr_b, sms_a, sms_b):
    cid = lax.axis_index("c")
    sid = lax.axis_index("s")
    wid = cid * NS + sid
    ebase = wid * _EPW
    souts = (s1_hbm, s2_hbm, s3_hbm, s4_hbm)

    idxs = (idx_a, idx_b)
    rows = (rows_a, rows_b)
    smi = (smi_a, smi_b)
    smr = (smr_a, smr_b)
    sms = (sms_a, sms_b)

    def load_start(i, b):
        pltpu.async_copy(src_hbm.at[wid, i], idxs[b], smi[b])
        pltpu.async_copy(msg_hbm.at[pl.ds(ebase + i * _CK, _CK)], rows[b],
                         smr[b])

    def load_wait(i, b):
        pltpu.make_async_copy(src_hbm.at[wid, i], idxs[b], smi[b]).wait()
        pltpu.make_async_copy(msg_hbm.at[pl.ds(ebase + i * _CK, _CK)],
                              rows[b], smr[b]).wait()

    def scat_start(b):
        pltpu.async_copy(rows[b], acc.at[idxs[b]], sms[b], add=True)

    def scat_wait(b):
        pltpu.make_async_copy(rows[b], acc.at[idxs[b]], sms[b]).wait()

    for p in range(4):
        for t in range(_TP):
            q = sid + NS * t

            @pl.when(q < _NPIECE)
            def _(q=q):
                pltpu.sync_copy(zero_hbm, acc.at[pl.ds(q * _PC, _PC)])

        plsc.subcore_barrier()

        def powers(buf, pass_id):
            if pass_id == 0:
                return

            def body(r, __):
                for gix in range(MP // LANES):
                    sl = pl.ds(gix * LANES, LANES)
                    v = buf[r, sl]
                    v2 = v * v
                    if pass_id == 1:
                        buf[r, sl] = v2
                    elif pass_id == 2:
                        buf[r, sl] = v2 * v
                    else:
                        buf[r, sl] = v2 * v2
                return 0

            lax.fori_loop(0, _CK, body, 0)

        load_start(0, 0)

        def pair(j, _, pass_id=p):
            i0 = 2 * j
            load_wait(i0, 0)

            @pl.when(j > 0)
            def _():
                scat_wait(1)

            load_start(i0 + 1, 1)
            powers(rows[0], pass_id)
            scat_start(0)

            i1 = i0 + 1
            load_wait(i1, 1)
            scat_wait(0)

            @pl.when(i1 + 1 < _NCH)
            def _():
                load_start(i1 + 1, 0)

            powers(rows[1], pass_id)
            scat_start(1)
            return 0

        lax.fori_loop(0, (_NCH - 1) // 2, pair, 0)
        i_last = _NCH - 1
        load_wait(i_last, 0)
        scat_wait(1)
        powers(rows[0], p)
        scat_start(0)
        scat_wait(0)
        plsc.subcore_barrier()

        for t in range(_TP):
            q = sid + NS * t

            @pl.when(q < _NPIECE)
            def _(q=q, p=p):
                rs = pl.ds(q * _PC, _PC)
                pltpu.sync_copy(acc.at[rs], souts[p].at[cid, rs])

        plsc.subcore_barrier()


def kernel(x_s, x_t, edge_index, edge_attr, x_u, W1, b1, W2, b2, W3, b3, W4,
           b4, g):
    src = edge_index[0].astype(jnp.int32).reshape(NW, _NCH, _CK)
    tgt = edge_index[1].astype(jnp.int32).reshape(NW, _NCH, _CK)

    W1a = W1[:DT]
    W1b = W1[DT:]
    W2p = jnp.pad(W2, ((0, 0), (0, MP - M)))
    b2p = jnp.concatenate(
        [b2, jnp.ones((1,), jnp.float32),
         jnp.zeros((MP - M - 1,), jnp.float32)]).reshape(1, MP)
    z16 = jnp.zeros((MP - M, U), jnp.float32)
    W3p = jnp.concatenate([
        W3[0:DS],
        W3[DS:DS + M], z16,
        W3[DS + M:DS + 2 * M], z16,
        W3[DS + 2 * M:DS + 3 * M], z16,
        W3[DS + 3 * M:DS + 4 * M], z16,
        W3[DS + 4 * M:U],
    ], axis=0)

    gathered = _sc_gather(x_t, tgt)
    msg = _edge_mlp(gathered, edge_attr, W1a, W1b, b1.reshape(1, M),
                    W2p, b2p)
    S1, S2, S3, S4 = _sc_moments(msg, src, jnp.zeros((_PC, MP), jnp.float32))
    return _node_mlp(x_s, S1, S2, S3, S4, x_u.reshape(1, GW),
                     W3p, b3.reshape(1, U), W4, b4.reshape(1, DS),
                     g.reshape(1, DS))
